# Initial kernel scaffold; baseline (speedup 1.0000x reference)
import jax
import jax.numpy as jnp
from jax.experimental import pallas as pl


def kernel(x, edge_index, batch, W1, b1, W2, b2, W3, b3):
    raise NotImplementedError("write your pallas kernel here")


# speedup vs baseline: 8.7783x; 8.7783x over previous
import functools

import jax
import jax.numpy as jnp
from jax import lax
from jax.experimental import pallas as pl
from jax.experimental.pallas import tpu as pltpu
from jax.experimental.pallas import tpu_sc as plsc

NN = 10000
EE = 160000
GG = 64
FF = 256
HH = 256
OO = 128

NC = 2
NS = 16
LANES = 16
CHUNK = 128
NCHUNKS = EE // CHUNK
RPS = 624
ROWBLK = 1000
NBLK = NN // ROWBLK

@functools.lru_cache(maxsize=None)
def _mesh():
    return plsc.VectorSubcoreMesh(
        core_axis_name="c", subcore_axis_name="s", num_cores=NC, num_subcores=NS)


def _zero_rows(buf, nrows, width):
    z = jnp.zeros((LANES,), jnp.float32)

    def body(r, _):
        for l in range(width // LANES):
            buf[r, pl.ds(l * LANES, LANES)] = z
        return 0

    lax.fori_loop(0, nrows, body, 0)


def _copy_span(src_at, dst_at, s):
    for off, sz in ((0, 128), (128, 128), (256, 128), (384, 128), (512, 112)):
        pltpu.sync_copy(src_at(off, sz), dst_at(off, sz))

    @pl.when(s == NS - 1)
    def _():
        pltpu.sync_copy(src_at(RPS, 16), dst_at(RPS, 16))


@functools.lru_cache(maxsize=None)
def _make_edge_scatter(half):
    @functools.partial(
        pl.kernel,
        out_type=jax.ShapeDtypeStruct((NC * NN, half), jnp.float32),
        mesh=_mesh(),
        scratch_types=[
            pltpu.VMEM_SHARED((NN, half), jnp.float32),
            pltpu.VMEM((CHUNK,), jnp.int32),
            pltpu.VMEM((CHUNK,), jnp.int32),
            pltpu.VMEM((CHUNK,), jnp.int32),
            pltpu.VMEM((CHUNK, half), jnp.float32),
        ],
    )
    def k(p_hbm, src_hbm, dst_hbm, out_hbm, acc, srcv, dstv, gidx, gbuf):
        c = lax.axis_index("c")
        s = lax.axis_index("s")

        _zero_rows(gbuf, CHUNK, half)
        base_row = s * RPS
        _copy_span(
            lambda off, sz: gbuf.at[pl.ds(0, sz)],
            lambda off, sz: acc.at[pl.ds(base_row + off, sz)], s)
        plsc.subcore_barrier()

        off_gather = c * NN
        nt = 78 + jnp.where(s < NCHUNKS - 78 * NS, 1, 0)

        def body(j, _):
            t = s + j * NS
            ebase = t * CHUNK
            pltpu.sync_copy(src_hbm.at[pl.ds(ebase, CHUNK)], srcv)
            pltpu.sync_copy(dst_hbm.at[pl.ds(ebase, CHUNK)], dstv)
            for r in range(CHUNK // LANES):
                sl = pl.ds(r * LANES, LANES)
                gidx[sl] = srcv[sl] + off_gather
            pltpu.sync_copy(p_hbm.at[gidx], gbuf)
            pltpu.sync_copy(gbuf, acc.at[dstv], add=True)
            return 0

        lax.fori_loop(0, nt, body, 0)
        plsc.subcore_barrier()

        out_base = c * NN + s * RPS
        _copy_span(
            lambda off, sz: acc.at[pl.ds(base_row + off, sz)],
            lambda off, sz: out_hbm.at[pl.ds(out_base + off, sz)], s)

    return k


@functools.lru_cache(maxsize=None)
def _make_edge_scatter_esplit(width):
    @functools.partial(
        pl.kernel,
        out_type=jax.ShapeDtypeStruct((NC * NN, width), jnp.float32),
        mesh=_mesh(),
        scratch_types=[
            pltpu.VMEM_SHARED((NN, width), jnp.float32),
            pltpu.VMEM((CHUNK,), jnp.int32),
            pltpu.VMEM((CHUNK,), jnp.int32),
            pltpu.VMEM((CHUNK, width), jnp.float32),
        ],
    )
    def k(p_hbm, src_hbm, dst_hbm, out_hbm, acc, srcv, dstv, gbuf):
        c = lax.axis_index("c")
        s = lax.axis_index("s")
        wid = c * NS + s

        _zero_rows(gbuf, CHUNK, width)
        base_row = s * RPS
        _copy_span(
            lambda off, sz: gbuf.at[pl.ds(0, sz)],
            lambda off, sz: acc.at[pl.ds(base_row + off, sz)], s)
        plsc.subcore_barrier()

        nt = 39 + jnp.where(wid < NCHUNKS - 39 * NC * NS, 1, 0)

        def body(j, _):
            t = wid + j * NC * NS
            ebase = t * CHUNK
            pltpu.sync_copy(src_hbm.at[pl.ds(ebase, CHUNK)], srcv)
            pltpu.sync_copy(dst_hbm.at[pl.ds(ebase, CHUNK)], dstv)
            pltpu.sync_copy(p_hbm.at[srcv], gbuf)
            pltpu.sync_copy(gbuf, acc.at[dstv], add=True)
            return 0

        lax.fori_loop(0, nt, body, 0)
        plsc.subcore_barrier()

        out_base = c * NN + s * RPS
        _copy_span(
            lambda off, sz: acc.at[pl.ds(base_row + off, sz)],
            lambda off, sz: out_hbm.at[pl.ds(out_base + off, sz)], s)

    return k


DEGW = 128


def _dinv_from_deg(deg_ref):
    dsum = deg_ref[0] + deg_ref[1]
    return lax.rsqrt(1.0 + dsum[:, 0:1])


def _split_out(out_ref, pn, half):
    out_ref[0, :, :] = pn[:, :half]
    out_ref[1, :, :] = pn[:, half:]


def _layer1_body(x_ref, deg_ref, w_ref, out_ref):
    dinv = _dinv_from_deg(deg_ref)
    m = jnp.dot(x_ref[...], w_ref[...], preferred_element_type=jnp.float32)
    _split_out(out_ref, dinv * m, HH // 2)


def _mid_body(s_ref, p_ref, deg_ref, b_ref, w_ref, out_ref, *, half_out):
    dinv = _dinv_from_deg(deg_ref)
    spp = jnp.concatenate(
        [s_ref[0] + p_ref[0], s_ref[1] + p_ref[1]], axis=-1)
    h = jnp.maximum(b_ref[...] + dinv * spp, 0.0)
    m = jnp.dot(h, w_ref[...], preferred_element_type=jnp.float32)
    pn = dinv * m
    if half_out is None:
        out_ref[...] = pn
    else:
        _split_out(out_ref, pn, half_out)


def _pool_body(s_ref, p_ref, deg_ref, b_ref, batch_ref, out_ref, cnt_ref):
    i = pl.program_id(0)

    @pl.when(i == 0)
    def _():
        out_ref[...] = jnp.zeros_like(out_ref)
        cnt_ref[...] = jnp.zeros_like(cnt_ref)

    dinv = _dinv_from_deg(deg_ref)
    spp = s_ref[0] + s_ref[1] + p_ref[...]
    h = b_ref[...] + dinv * spp
    bid = jnp.broadcast_to(batch_ref[0], (GG, ROWBLK))
    gid = lax.broadcasted_iota(jnp.int32, (GG, ROWBLK), 0)
    oh = (bid == gid).astype(jnp.float32)
    out_ref[...] += jnp.dot(oh, h, preferred_element_type=jnp.float32)
    cnt_ref[...] += jnp.broadcast_to(
        jnp.sum(oh, axis=1, keepdims=True), (GG, OO))

    @pl.when(i == NBLK - 1)
    def _():
        out_ref[...] = out_ref[...] / jnp.maximum(cnt_ref[...], 1.0)


def _rowspec(shape3):
    return pl.BlockSpec((shape3[0], ROWBLK, shape3[2]), lambda i: (0, i, 0))


def _full(shape):
    return pl.BlockSpec(shape, lambda i: tuple(0 for _ in shape))


_layer1 = pl.pallas_call(
    _layer1_body,
    grid=(NBLK,),
    in_specs=[
        pl.BlockSpec((ROWBLK, FF), lambda i: (i, 0)),
        _rowspec((NC, NN, DEGW)),
        _full((FF, HH)),
    ],
    out_specs=_rowspec((NC, NN, HH // 2)),
    out_shape=jax.ShapeDtypeStruct((NC, NN, HH // 2), jnp.float32),
)

_layer2 = pl.pallas_call(
    functools.partial(_mid_body, half_out=HH // 2),
    grid=(NBLK,),
    in_specs=[
        _rowspec((NC, NN, HH // 2)),
        _rowspec((NC, NN, HH // 2)),
        _rowspec((NC, NN, DEGW)),
        _full((1, HH)),
        _full((HH, HH)),
    ],
    out_specs=_rowspec((NC, NN, HH // 2)),
    out_shape=jax.ShapeDtypeStruct((NC, NN, HH // 2), jnp.float32),
)

_layer3 = pl.pallas_call(
    functools.partial(_mid_body, half_out=None),
    grid=(NBLK,),
    in_specs=[
        _rowspec((NC, NN, HH // 2)),
        _rowspec((NC, NN, HH // 2)),
        _rowspec((NC, NN, DEGW)),
        _full((1, HH)),
        _full((HH, OO)),
    ],
    out_specs=pl.BlockSpec((ROWBLK, OO), lambda i: (i, 0)),
    out_shape=jax.ShapeDtypeStruct((NN, OO), jnp.float32),
)

_pool = pl.pallas_call(
    _pool_body,
    grid=(NBLK,),
    in_specs=[
        _rowspec((NC, NN, OO)),
        pl.BlockSpec((ROWBLK, OO), lambda i: (i, 0)),
        _rowspec((NC, NN, DEGW)),
        _full((1, OO)),
        pl.BlockSpec((1, 1, ROWBLK), lambda i: (i, 0, 0)),
    ],
    out_specs=_full((GG, OO)),
    out_shape=jax.ShapeDtypeStruct((GG, OO), jnp.float32),
    scratch_shapes=[pltpu.VMEM((GG, OO), jnp.float32)],
)


def kernel(x, edge_index, batch, W1, b1, W2, b2, W3, b3):
    src = edge_index[0]
    dst = edge_index[1]
    scat128 = _make_edge_scatter(HH // 2)
    scat_last = _make_edge_scatter_esplit(OO)
    ones = jnp.ones((NN, DEGW), jnp.float32)
    degs = scat_last(ones, src, dst).reshape(NC, NN, DEGW)
    p1 = _layer1(x, degs, W1)
    s1 = scat128(p1.reshape(NC * NN, HH // 2), src, dst).reshape(NC, NN, HH // 2)
    p2 = _layer2(s1, p1, degs, b1.reshape(1, HH), W2)
    s2 = scat128(p2.reshape(NC * NN, HH // 2), src, dst).reshape(NC, NN, HH // 2)
    p3 = _layer3(s2, p2, degs, b2.reshape(1, HH), W3)
    s3 = scat_last(p3, src, dst).reshape(NC, NN, OO)
    return _pool(s3, p3, degs, b3.reshape(1, OO), batch.reshape(NBLK, 1, ROWBLK))


# speedup vs baseline: 14.7303x; 1.6780x over previous
import functools

import jax
import jax.numpy as jnp
from jax import lax
from jax.experimental import pallas as pl
from jax.experimental.pallas import tpu as pltpu
from jax.experimental.pallas import tpu_sc as plsc

NN = 10000
EE = 160000
GG = 64
FF = 256
HH = 256
OO = 128

NC = 2
NS = 16
LANES = 16
CHUNK = 128
NCHUNKS = EE // CHUNK
RPS = 624
ROWBLK = 1000
NBLK = NN // ROWBLK

@functools.lru_cache(maxsize=None)
def _mesh():
    return plsc.VectorSubcoreMesh(
        core_axis_name="c", subcore_axis_name="s", num_cores=NC, num_subcores=NS)


def _zero_rows(buf, nrows, width):
    z = jnp.zeros((LANES,), jnp.float32)

    def body(r, _):
        for l in range(width // LANES):
            buf[r, pl.ds(l * LANES, LANES)] = z
        return 0

    lax.fori_loop(0, nrows, body, 0)


def _copy_span(src_at, dst_at, s):
    for off, sz in ((0, 128), (128, 128), (256, 128), (384, 128), (512, 112)):
        pltpu.sync_copy(src_at(off, sz), dst_at(off, sz))

    @pl.when(s == NS - 1)
    def _():
        pltpu.sync_copy(src_at(RPS, 16), dst_at(RPS, 16))


@functools.lru_cache(maxsize=None)
def _make_edge_scatter(half):
    @functools.partial(
        pl.kernel,
        out_type=jax.ShapeDtypeStruct((NC * NN, half), jnp.float32),
        mesh=_mesh(),
        scratch_types=[
            pltpu.VMEM_SHARED((NN, half), jnp.float32),
            pltpu.VMEM((CHUNK,), jnp.int32),
            pltpu.VMEM((CHUNK,), jnp.int32),
            pltpu.VMEM((CHUNK,), jnp.int32),
            pltpu.VMEM((CHUNK,), jnp.int32),
            pltpu.VMEM((CHUNK,), jnp.int32),
            pltpu.VMEM((CHUNK,), jnp.int32),
            pltpu.VMEM((CHUNK, half), jnp.float32),
            pltpu.VMEM((CHUNK, half), jnp.float32),
            pltpu.SemaphoreType.DMA,
            pltpu.SemaphoreType.DMA,
        ],
    )
    def k(p_hbm, src_hbm, dst_hbm, out_hbm, acc,
          srcv0, srcv1, dstv0, dstv1, gidx0, gidx1, gbuf0, gbuf1, sem0, sem1):
        c = lax.axis_index("c")
        s = lax.axis_index("s")
        bufs = ((srcv0, dstv0, gidx0, gbuf0, sem0),
                (srcv1, dstv1, gidx1, gbuf1, sem1))

        _zero_rows(gbuf0, CHUNK, half)
        base_row = s * RPS
        _copy_span(
            lambda off, sz: gbuf0.at[pl.ds(0, sz)],
            lambda off, sz: acc.at[pl.ds(base_row + off, sz)], s)
        plsc.subcore_barrier()

        off_gather = c * NN
        nt = 78 + jnp.where(s < NCHUNKS - 78 * NS, 1, 0)

        def load_and_fire(kk, bi):
            sv, dv, gi, gb, sm = bufs[bi]
            ebase = (s + kk * NS) * CHUNK
            pltpu.sync_copy(src_hbm.at[pl.ds(ebase, CHUNK)], sv)
            pltpu.sync_copy(dst_hbm.at[pl.ds(ebase, CHUNK)], dv)
            for r in range(CHUNK // LANES):
                sl = pl.ds(r * LANES, LANES)
                gi[sl] = sv[sl] + off_gather
            pltpu.async_copy(p_hbm.at[gi], gb, sm)

        load_and_fire(0, 0)

        def body(j2, _):
            for b2 in (0, 1):
                kk = j2 * 2 + b2
                sv, dv, gi, gb, sm = bufs[b2]

                @pl.when(kk + 1 < nt)
                def _():
                    load_and_fire(kk + 1, 1 - b2)

                @pl.when(kk < nt)
                def _():
                    pltpu.make_async_copy(p_hbm.at[gi], gb, sm).wait()
                    pltpu.sync_copy(gb, acc.at[dv], add=True)
            return 0

        lax.fori_loop(0, 40, body, 0)
        plsc.subcore_barrier()

        out_base = c * NN + s * RPS
        _copy_span(
            lambda off, sz: acc.at[pl.ds(base_row + off, sz)],
            lambda off, sz: out_hbm.at[pl.ds(out_base + off, sz)], s)

    return k


@functools.lru_cache(maxsize=None)
def _make_edge_scatter_esplit(width):
    @functools.partial(
        pl.kernel,
        out_type=jax.ShapeDtypeStruct((NC * NN, width), jnp.float32),
        mesh=_mesh(),
        scratch_types=[
            pltpu.VMEM_SHARED((NN, width), jnp.float32),
            pltpu.VMEM((CHUNK,), jnp.int32),
            pltpu.VMEM((CHUNK,), jnp.int32),
            pltpu.VMEM((CHUNK,), jnp.int32),
            pltpu.VMEM((CHUNK,), jnp.int32),
            pltpu.VMEM((CHUNK, width), jnp.float32),
            pltpu.VMEM((CHUNK, width), jnp.float32),
            pltpu.SemaphoreType.DMA,
            pltpu.SemaphoreType.DMA,
        ],
    )
    def k(p_hbm, src_hbm, dst_hbm, out_hbm, acc,
          srcv0, srcv1, dstv0, dstv1, gbuf0, gbuf1, sem0, sem1):
        c = lax.axis_index("c")
        s = lax.axis_index("s")
        wid = c * NS + s
        bufs = ((srcv0, dstv0, gbuf0, sem0), (srcv1, dstv1, gbuf1, sem1))

        _zero_rows(gbuf0, CHUNK, width)
        base_row = s * RPS
        _copy_span(
            lambda off, sz: gbuf0.at[pl.ds(0, sz)],
            lambda off, sz: acc.at[pl.ds(base_row + off, sz)], s)
        plsc.subcore_barrier()

        nt = 39 + jnp.where(wid < NCHUNKS - 39 * NC * NS, 1, 0)

        def load_and_fire(kk, bi):
            sv, dv, gb, sm = bufs[bi]
            ebase = (wid + kk * NC * NS) * CHUNK
            pltpu.sync_copy(src_hbm.at[pl.ds(ebase, CHUNK)], sv)
            pltpu.sync_copy(dst_hbm.at[pl.ds(ebase, CHUNK)], dv)
            pltpu.async_copy(p_hbm.at[sv], gb, sm)

        load_and_fire(0, 0)

        def body(j2, _):
            for b2 in (0, 1):
                kk = j2 * 2 + b2
                sv, dv, gb, sm = bufs[b2]

                @pl.when(kk + 1 < nt)
                def _():
                    load_and_fire(kk + 1, 1 - b2)

                @pl.when(kk < nt)
                def _():
                    pltpu.make_async_copy(p_hbm.at[sv], gb, sm).wait()
                    pltpu.sync_copy(gb, acc.at[dv], add=True)
            return 0

        lax.fori_loop(0, 20, body, 0)
        plsc.subcore_barrier()

        out_base = c * NN + s * RPS
        _copy_span(
            lambda off, sz: acc.at[pl.ds(base_row + off, sz)],
            lambda off, sz: out_hbm.at[pl.ds(out_base + off, sz)], s)

    return k


@functools.lru_cache(maxsize=None)
def _make_deg_kernel():
    width = DEGW

    @functools.partial(
        pl.kernel,
        out_type=jax.ShapeDtypeStruct((NC * NN, width), jnp.float32),
        mesh=_mesh(),
        scratch_types=[
            pltpu.VMEM_SHARED((NN, width), jnp.float32),
            pltpu.VMEM((CHUNK,), jnp.int32),
            pltpu.VMEM((CHUNK,), jnp.int32),
            pltpu.VMEM((CHUNK, width), jnp.float32),
            pltpu.VMEM((CHUNK, width), jnp.float32),
            pltpu.SemaphoreType.DMA,
            pltpu.SemaphoreType.DMA,
        ],
    )
    def k(dst_hbm, out_hbm, acc, dstv0, dstv1, zerob, onesb, sem0, sem1):
        c = lax.axis_index("c")
        s = lax.axis_index("s")
        wid = c * NS + s
        bufs = ((dstv0, sem0), (dstv1, sem1))

        _zero_rows(zerob, CHUNK, width)
        one = jnp.ones((LANES,), jnp.float32)

        def fill(r, _):
            for l in range(width // LANES):
                onesb[r, pl.ds(l * LANES, LANES)] = one
            return 0

        lax.fori_loop(0, CHUNK, fill, 0)

        base_row = s * RPS
        _copy_span(
            lambda off, sz: zerob.at[pl.ds(0, sz)],
            lambda off, sz: acc.at[pl.ds(base_row + off, sz)], s)
        plsc.subcore_barrier()

        nt = 39 + jnp.where(wid < NCHUNKS - 39 * NC * NS, 1, 0)

        def body(j2, _):
            for b2 in (0, 1):
                kk = j2 * 2 + b2
                dv, sm = bufs[b2]

                @pl.when(kk < nt)
                def _():
                    @pl.when(kk >= 2)
                    def _():
                        pltpu.make_async_copy(onesb, acc.at[dv], sm).wait()

                    ebase = (wid + kk * NC * NS) * CHUNK
                    pltpu.sync_copy(dst_hbm.at[pl.ds(ebase, CHUNK)], dv)
                    pltpu.async_copy(onesb, acc.at[dv], sm, add=True)
            return 0

        lax.fori_loop(0, 20, body, 0)
        for b2 in (0, 1):
            dv, sm = bufs[b2]
            pltpu.make_async_copy(onesb, acc.at[dv], sm).wait()
        plsc.subcore_barrier()

        out_base = c * NN + s * RPS
        _copy_span(
            lambda off, sz: acc.at[pl.ds(base_row + off, sz)],
            lambda off, sz: out_hbm.at[pl.ds(out_base + off, sz)], s)

    return k


DEGW = 128


def _dinv_from_deg(deg_ref):
    dsum = deg_ref[0] + deg_ref[1]
    return lax.rsqrt(1.0 + dsum[:, 0:1])


def _split_out(out_ref, pn, half):
    out_ref[0, :, :] = pn[:, :half]
    out_ref[1, :, :] = pn[:, half:]


def _layer1_body(x_ref, deg_ref, w_ref, out_ref):
    dinv = _dinv_from_deg(deg_ref)
    m = jnp.dot(x_ref[...], w_ref[...], preferred_element_type=jnp.float32)
    _split_out(out_ref, dinv * m, HH // 2)


def _mid_body(s_ref, p_ref, deg_ref, b_ref, w_ref, out_ref, *, half_out):
    dinv = _dinv_from_deg(deg_ref)
    spp = jnp.concatenate(
        [s_ref[0] + p_ref[0], s_ref[1] + p_ref[1]], axis=-1)
    h = jnp.maximum(b_ref[...] + dinv * spp, 0.0)
    m = jnp.dot(h, w_ref[...], preferred_element_type=jnp.float32)
    pn = dinv * m
    if half_out is None:
        out_ref[...] = pn
    else:
        _split_out(out_ref, pn, half_out)


def _pool_body(s_ref, p_ref, deg_ref, b_ref, batch_ref, out_ref, cnt_ref):
    i = pl.program_id(0)

    @pl.when(i == 0)
    def _():
        out_ref[...] = jnp.zeros_like(out_ref)
        cnt_ref[...] = jnp.zeros_like(cnt_ref)

    dinv = _dinv_from_deg(deg_ref)
    spp = s_ref[0] + s_ref[1] + p_ref[...]
    h = b_ref[...] + dinv * spp
    bid = jnp.broadcast_to(batch_ref[0], (GG, ROWBLK))
    gid = lax.broadcasted_iota(jnp.int32, (GG, ROWBLK), 0)
    oh = (bid == gid).astype(jnp.float32)
    out_ref[...] += jnp.dot(oh, h, preferred_element_type=jnp.float32)
    cnt_ref[...] += jnp.broadcast_to(
        jnp.sum(oh, axis=1, keepdims=True), (GG, OO))

    @pl.when(i == NBLK - 1)
    def _():
        out_ref[...] = out_ref[...] / jnp.maximum(cnt_ref[...], 1.0)


def _rowspec(shape3):
    return pl.BlockSpec((shape3[0], ROWBLK, shape3[2]), lambda i: (0, i, 0))


def _full(shape):
    return pl.BlockSpec(shape, lambda i: tuple(0 for _ in shape))


_layer1 = pl.pallas_call(
    _layer1_body,
    grid=(NBLK,),
    in_specs=[
        pl.BlockSpec((ROWBLK, FF), lambda i: (i, 0)),
        _rowspec((NC, NN, DEGW)),
        _full((FF, HH)),
    ],
    out_specs=_rowspec((NC, NN, HH // 2)),
    out_shape=jax.ShapeDtypeStruct((NC, NN, HH // 2), jnp.float32),
)

_layer2 = pl.pallas_call(
    functools.partial(_mid_body, half_out=HH // 2),
    grid=(NBLK,),
    in_specs=[
        _rowspec((NC, NN, HH // 2)),
        _rowspec((NC, NN, HH // 2)),
        _rowspec((NC, NN, DEGW)),
        _full((1, HH)),
        _full((HH, HH)),
    ],
    out_specs=_rowspec((NC, NN, HH // 2)),
    out_shape=jax.ShapeDtypeStruct((NC, NN, HH // 2), jnp.float32),
)

_layer3 = pl.pallas_call(
    functools.partial(_mid_body, half_out=None),
    grid=(NBLK,),
    in_specs=[
        _rowspec((NC, NN, HH // 2)),
        _rowspec((NC, NN, HH // 2)),
        _rowspec((NC, NN, DEGW)),
        _full((1, HH)),
        _full((HH, OO)),
    ],
    out_specs=pl.BlockSpec((ROWBLK, OO), lambda i: (i, 0)),
    out_shape=jax.ShapeDtypeStruct((NN, OO), jnp.float32),
)

_pool = pl.pallas_call(
    _pool_body,
    grid=(NBLK,),
    in_specs=[
        _rowspec((NC, NN, OO)),
        pl.BlockSpec((ROWBLK, OO), lambda i: (i, 0)),
        _rowspec((NC, NN, DEGW)),
        _full((1, OO)),
        pl.BlockSpec((1, 1, ROWBLK), lambda i: (i, 0, 0)),
    ],
    out_specs=_full((GG, OO)),
    out_shape=jax.ShapeDtypeStruct((GG, OO), jnp.float32),
    scratch_shapes=[pltpu.VMEM((GG, OO), jnp.float32)],
)


def kernel(x, edge_index, batch, W1, b1, W2, b2, W3, b3):
    src = edge_index[0]
    dst = edge_index[1]
    scat128 = _make_edge_scatter(HH // 2)
    scat_last = _make_edge_scatter_esplit(OO)
    degs = _make_deg_kernel()(dst).reshape(NC, NN, DEGW)
    p1 = _layer1(x, degs, W1)
    s1 = scat128(p1.reshape(NC * NN, HH // 2), src, dst).reshape(NC, NN, HH // 2)
    p2 = _layer2(s1, p1, degs, b1.reshape(1, HH), W2)
    s2 = scat128(p2.reshape(NC * NN, HH // 2), src, dst).reshape(NC, NN, HH // 2)
    p3 = _layer3(s2, p2, degs, b2.reshape(1, HH), W3)
    s3 = scat_last(p3, src, dst).reshape(NC, NN, OO)
    return _pool(s3, p3, degs, b3.reshape(1, OO), batch.reshape(NBLK, 1, ROWBLK))


# speedup vs baseline: 16.1526x; 1.0966x over previous
import functools

import jax
import jax.numpy as jnp
from jax import lax
from jax.experimental import pallas as pl
from jax.experimental.pallas import tpu as pltpu
from jax.experimental.pallas import tpu_sc as plsc

NN = 10000
EE = 160000
GG = 64
FF = 256
HH = 256
OO = 128

NC = 2
NS = 16
LANES = 16
CHUNK = 128
NCHUNKS = EE // CHUNK
GSZ = 8
NG = 157
NIDX = NG * GSZ
RPS = 624
ROWBLK = 1000
NBLK = NN // ROWBLK

@functools.lru_cache(maxsize=None)
def _mesh():
    return plsc.VectorSubcoreMesh(
        core_axis_name="c", subcore_axis_name="s", num_cores=NC, num_subcores=NS)


def _zero_rows(buf, nrows, width):
    z = jnp.zeros((LANES,), jnp.float32)

    def body(r, _):
        for l in range(width // LANES):
            buf[r, pl.ds(l * LANES, LANES)] = z
        return 0

    lax.fori_loop(0, nrows, body, 0)


def _copy_span(src_at, dst_at, s):
    for off, sz in ((0, 128), (128, 128), (256, 128), (384, 128), (512, 112)):
        pltpu.sync_copy(src_at(off, sz), dst_at(off, sz))

    @pl.when(s == NS - 1)
    def _():
        pltpu.sync_copy(src_at(RPS, 16), dst_at(RPS, 16))


@functools.lru_cache(maxsize=None)
def _make_edge_scatter(half):
    @functools.partial(
        pl.kernel,
        out_type=jax.ShapeDtypeStruct((NC * NN, half), jnp.float32),
        mesh=_mesh(),
        scratch_types=[
            pltpu.VMEM_SHARED((NN, half), jnp.float32),
            pltpu.VMEM((GSZ, CHUNK), jnp.int32),
            pltpu.VMEM((GSZ, CHUNK), jnp.int32),
            pltpu.VMEM((GSZ, CHUNK), jnp.int32),
            pltpu.VMEM((CHUNK, half), jnp.float32),
            pltpu.VMEM((CHUNK, half), jnp.float32),
            pltpu.SemaphoreType.DMA,
            pltpu.SemaphoreType.DMA,
        ],
    )
    def k(p_hbm, src_hbm, dst_hbm, out_hbm, acc,
          sidx, didx, gidx, gbuf0, gbuf1, sem0, sem1):
        c = lax.axis_index("c")
        s = lax.axis_index("s")
        gbufs = ((gbuf0, sem0), (gbuf1, sem1))

        _zero_rows(gbuf0, CHUNK, half)
        base_row = s * RPS
        _copy_span(
            lambda off, sz: gbuf0.at[pl.ds(0, sz)],
            lambda off, sz: acc.at[pl.ds(base_row + off, sz)], s)
        plsc.subcore_barrier()

        off_gather = c * NN
        ngrp = 9 + jnp.where(s < NG - 9 * NS, 1, 0)

        def grp(j, _):
            g = s + j * NS
            pltpu.sync_copy(src_hbm.at[pl.ds(g * GSZ, GSZ)], sidx)
            pltpu.sync_copy(dst_hbm.at[pl.ds(g * GSZ, GSZ)], didx)
            for i in range(GSZ):
                for r in range(CHUNK // LANES):
                    sl = pl.ds(r * LANES, LANES)
                    gidx[i, sl] = sidx[i, sl] + off_gather
            t0 = g * GSZ

            def fire(i):
                gb, sm = gbufs[i % 2]
                pltpu.async_copy(p_hbm.at[gidx.at[i]], gb, sm)

            @pl.when(t0 < NCHUNKS)
            def _():
                fire(0)

            for i in range(GSZ):
                if i + 1 < GSZ:
                    @pl.when(t0 + i + 1 < NCHUNKS)
                    def _(i=i):
                        fire(i + 1)

                gb, sm = gbufs[i % 2]

                @pl.when(t0 + i < NCHUNKS)
                def _(i=i, gb=gb, sm=sm):
                    pltpu.make_async_copy(p_hbm.at[gidx.at[i]], gb, sm).wait()
                    pltpu.sync_copy(gb, acc.at[didx.at[i]], add=True)
            return 0

        lax.fori_loop(0, ngrp, grp, 0)
        plsc.subcore_barrier()

        out_base = c * NN + s * RPS
        _copy_span(
            lambda off, sz: acc.at[pl.ds(base_row + off, sz)],
            lambda off, sz: out_hbm.at[pl.ds(out_base + off, sz)], s)

    return k


@functools.lru_cache(maxsize=None)
def _make_edge_scatter_esplit(width):
    @functools.partial(
        pl.kernel,
        out_type=jax.ShapeDtypeStruct((NC * NN, width), jnp.float32),
        mesh=_mesh(),
        scratch_types=[
            pltpu.VMEM_SHARED((NN, width), jnp.float32),
            pltpu.VMEM((GSZ, CHUNK), jnp.int32),
            pltpu.VMEM((GSZ, CHUNK), jnp.int32),
            pltpu.VMEM((CHUNK, width), jnp.float32),
            pltpu.VMEM((CHUNK, width), jnp.float32),
            pltpu.SemaphoreType.DMA,
            pltpu.SemaphoreType.DMA,
        ],
    )
    def k(p_hbm, src_hbm, dst_hbm, out_hbm, acc,
          sidx, didx, gbuf0, gbuf1, sem0, sem1):
        c = lax.axis_index("c")
        s = lax.axis_index("s")
        wid = c * NS + s
        gbufs = ((gbuf0, sem0), (gbuf1, sem1))

        _zero_rows(gbuf0, CHUNK, width)
        base_row = s * RPS
        _copy_span(
            lambda off, sz: gbuf0.at[pl.ds(0, sz)],
            lambda off, sz: acc.at[pl.ds(base_row + off, sz)], s)
        plsc.subcore_barrier()

        ngrp = 4 + jnp.where(wid < NG - 4 * NC * NS, 1, 0)

        def grp(j, _):
            g = wid + j * NC * NS
            pltpu.sync_copy(src_hbm.at[pl.ds(g * GSZ, GSZ)], sidx)
            pltpu.sync_copy(dst_hbm.at[pl.ds(g * GSZ, GSZ)], didx)
            t0 = g * GSZ

            def fire(i):
                gb, sm = gbufs[i % 2]
                pltpu.async_copy(p_hbm.at[sidx.at[i]], gb, sm)

            @pl.when(t0 < NCHUNKS)
            def _():
                fire(0)

            for i in range(GSZ):
                if i + 1 < GSZ:
                    @pl.when(t0 + i + 1 < NCHUNKS)
                    def _(i=i):
                        fire(i + 1)

                gb, sm = gbufs[i % 2]

                @pl.when(t0 + i < NCHUNKS)
                def _(i=i, gb=gb, sm=sm):
                    pltpu.make_async_copy(p_hbm.at[sidx.at[i]], gb, sm).wait()
                    pltpu.sync_copy(gb, acc.at[didx.at[i]], add=True)
            return 0

        lax.fori_loop(0, ngrp, grp, 0)
        plsc.subcore_barrier()

        out_base = c * NN + s * RPS
        _copy_span(
            lambda off, sz: acc.at[pl.ds(base_row + off, sz)],
            lambda off, sz: out_hbm.at[pl.ds(out_base + off, sz)], s)

    return k


@functools.lru_cache(maxsize=None)
def _make_deg_kernel():
    width = DEGW

    @functools.partial(
        pl.kernel,
        out_type=jax.ShapeDtypeStruct((NC * NN, width), jnp.float32),
        mesh=_mesh(),
        scratch_types=[
            pltpu.VMEM_SHARED((NN, width), jnp.float32),
            pltpu.VMEM((CHUNK,), jnp.int32),
            pltpu.VMEM((CHUNK,), jnp.int32),
            pltpu.VMEM((CHUNK, width), jnp.float32),
            pltpu.VMEM((CHUNK, width), jnp.float32),
            pltpu.SemaphoreType.DMA,
            pltpu.SemaphoreType.DMA,
        ],
    )
    def k(dst_hbm, out_hbm, acc, dstv0, dstv1, zerob, onesb, sem0, sem1):
        c = lax.axis_index("c")
        s = lax.axis_index("s")
        wid = c * NS + s
        bufs = ((dstv0, sem0), (dstv1, sem1))

        _zero_rows(zerob, CHUNK, width)
        one = jnp.ones((LANES,), jnp.float32)

        def fill(r, _):
            for l in range(width // LANES):
                onesb[r, pl.ds(l * LANES, LANES)] = one
            return 0

        lax.fori_loop(0, CHUNK, fill, 0)

        base_row = s * RPS
        _copy_span(
            lambda off, sz: zerob.at[pl.ds(0, sz)],
            lambda off, sz: acc.at[pl.ds(base_row + off, sz)], s)
        plsc.subcore_barrier()

        nt = 39 + jnp.where(wid < NCHUNKS - 39 * NC * NS, 1, 0)

        def body(j2, _):
            for b2 in (0, 1):
                kk = j2 * 2 + b2
                dv, sm = bufs[b2]

                @pl.when(kk < nt)
                def _():
                    @pl.when(kk >= 2)
                    def _():
                        pltpu.make_async_copy(onesb, acc.at[dv], sm).wait()

                    ebase = (wid + kk * NC * NS) * CHUNK
                    pltpu.sync_copy(dst_hbm.at[pl.ds(ebase, CHUNK)], dv)
                    pltpu.async_copy(onesb, acc.at[dv], sm, add=True)
            return 0

        lax.fori_loop(0, 20, body, 0)
        for b2 in (0, 1):
            dv, sm = bufs[b2]
            pltpu.make_async_copy(onesb, acc.at[dv], sm).wait()
        plsc.subcore_barrier()

        out_base = c * NN + s * RPS
        _copy_span(
            lambda off, sz: acc.at[pl.ds(base_row + off, sz)],
            lambda off, sz: out_hbm.at[pl.ds(out_base + off, sz)], s)

    return k


DEGW = 128


def _dinv_from_deg(deg_ref):
    dsum = deg_ref[0] + deg_ref[1]
    return lax.rsqrt(1.0 + dsum[:, 0:1])


def _split_out(out_ref, pn, half):
    out_ref[0, :, :] = pn[:, :half]
    out_ref[1, :, :] = pn[:, half:]


def _layer1_body(x_ref, deg_ref, w_ref, out_ref):
    dinv = _dinv_from_deg(deg_ref)
    m = jnp.dot(x_ref[...], w_ref[...], preferred_element_type=jnp.float32)
    _split_out(out_ref, dinv * m, HH // 2)


def _mid_body(s_ref, p_ref, deg_ref, b_ref, w_ref, out_ref, *, half_out):
    dinv = _dinv_from_deg(deg_ref)
    spp = jnp.concatenate(
        [s_ref[0] + p_ref[0], s_ref[1] + p_ref[1]], axis=-1)
    h = jnp.maximum(b_ref[...] + dinv * spp, 0.0)
    m = jnp.dot(h, w_ref[...], preferred_element_type=jnp.float32)
    pn = dinv * m
    if half_out is None:
        out_ref[...] = pn
    else:
        _split_out(out_ref, pn, half_out)


def _pool_body(s_ref, p_ref, deg_ref, b_ref, batch_ref, out_ref, cnt_ref):
    i = pl.program_id(0)

    @pl.when(i == 0)
    def _():
        out_ref[...] = jnp.zeros_like(out_ref)
        cnt_ref[...] = jnp.zeros_like(cnt_ref)

    dinv = _dinv_from_deg(deg_ref)
    spp = s_ref[0] + s_ref[1] + p_ref[...]
    h = b_ref[...] + dinv * spp
    bid = jnp.broadcast_to(batch_ref[0], (GG, ROWBLK))
    gid = lax.broadcasted_iota(jnp.int32, (GG, ROWBLK), 0)
    oh = (bid == gid).astype(jnp.float32)
    out_ref[...] += jnp.dot(oh, h, preferred_element_type=jnp.float32)
    cnt_ref[...] += jnp.broadcast_to(
        jnp.sum(oh, axis=1, keepdims=True), (GG, OO))

    @pl.when(i == NBLK - 1)
    def _():
        out_ref[...] = out_ref[...] / jnp.maximum(cnt_ref[...], 1.0)


def _rowspec(shape3):
    return pl.BlockSpec((shape3[0], ROWBLK, shape3[2]), lambda i: (0, i, 0))


def _full(shape):
    return pl.BlockSpec(shape, lambda i: tuple(0 for _ in shape))


_layer1 = pl.pallas_call(
    _layer1_body,
    grid=(NBLK,),
    in_specs=[
        pl.BlockSpec((ROWBLK, FF), lambda i: (i, 0)),
        _rowspec((NC, NN, DEGW)),
        _full((FF, HH)),
    ],
    out_specs=_rowspec((NC, NN, HH // 2)),
    out_shape=jax.ShapeDtypeStruct((NC, NN, HH // 2), jnp.float32),
)

_layer2 = pl.pallas_call(
    functools.partial(_mid_body, half_out=HH // 2),
    grid=(NBLK,),
    in_specs=[
        _rowspec((NC, NN, HH // 2)),
        _rowspec((NC, NN, HH // 2)),
        _rowspec((NC, NN, DEGW)),
        _full((1, HH)),
        _full((HH, HH)),
    ],
    out_specs=_rowspec((NC, NN, HH // 2)),
    out_shape=jax.ShapeDtypeStruct((NC, NN, HH // 2), jnp.float32),
)

_layer3 = pl.pallas_call(
    functools.partial(_mid_body, half_out=None),
    grid=(NBLK,),
    in_specs=[
        _rowspec((NC, NN, HH // 2)),
        _rowspec((NC, NN, HH // 2)),
        _rowspec((NC, NN, DEGW)),
        _full((1, HH)),
        _full((HH, OO)),
    ],
    out_specs=pl.BlockSpec((ROWBLK, OO), lambda i: (i, 0)),
    out_shape=jax.ShapeDtypeStruct((NN, OO), jnp.float32),
)

_pool = pl.pallas_call(
    _pool_body,
    grid=(NBLK,),
    in_specs=[
        _rowspec((NC, NN, OO)),
        pl.BlockSpec((ROWBLK, OO), lambda i: (i, 0)),
        _rowspec((NC, NN, DEGW)),
        _full((1, OO)),
        pl.BlockSpec((1, 1, ROWBLK), lambda i: (i, 0, 0)),
    ],
    out_specs=_full((GG, OO)),
    out_shape=jax.ShapeDtypeStruct((GG, OO), jnp.float32),
    scratch_shapes=[pltpu.VMEM((GG, OO), jnp.float32)],
)


def _pad_idx(a):
    pad = jnp.zeros((NIDX * CHUNK - EE,), jnp.int32)
    return jnp.concatenate([a, pad]).reshape(NIDX, CHUNK)


def kernel(x, edge_index, batch, W1, b1, W2, b2, W3, b3):
    src = edge_index[0]
    dst = edge_index[1]
    srcp = _pad_idx(src)
    dstp = _pad_idx(dst)
    scat128 = _make_edge_scatter(HH // 2)
    scat_last = _make_edge_scatter_esplit(OO)
    degs = _make_deg_kernel()(dst).reshape(NC, NN, DEGW)
    p1 = _layer1(x, degs, W1)
    s1 = scat128(p1.reshape(NC * NN, HH // 2), srcp, dstp).reshape(NC, NN, HH // 2)
    p2 = _layer2(s1, p1, degs, b1.reshape(1, HH), W2)
    s2 = scat128(p2.reshape(NC * NN, HH // 2), srcp, dstp).reshape(NC, NN, HH // 2)
    p3 = _layer3(s2, p2, degs, b2.reshape(1, HH), W3)
    s3 = scat_last(p3, srcp, dstp).reshape(NC, NN, OO)
    return _pool(s3, p3, degs, b3.reshape(1, OO), batch.reshape(NBLK, 1, ROWBLK))


# speedup vs baseline: 16.3084x; 1.0097x over previous
import functools

import jax
import jax.numpy as jnp
from jax import lax
from jax.experimental import pallas as pl
from jax.experimental.pallas import tpu as pltpu
from jax.experimental.pallas import tpu_sc as plsc

NN = 10000
EE = 160000
GG = 64
FF = 256
HH = 256
OO = 128

NC = 2
NS = 16
LANES = 16
CHUNK = 64
NCHUNKS = EE // CHUNK
GSZ = 16
NG = 157
NRING = 4
NIDX = NG * GSZ
RPS = 624
ROWBLK = 1000
NBLK = NN // ROWBLK

@functools.lru_cache(maxsize=None)
def _mesh():
    return plsc.VectorSubcoreMesh(
        core_axis_name="c", subcore_axis_name="s", num_cores=NC, num_subcores=NS)


def _zero_rows(buf, nrows, width):
    z = jnp.zeros((LANES,), jnp.float32)

    def body(r, _):
        for l in range(width // LANES):
            buf[r, pl.ds(l * LANES, LANES)] = z
        return 0

    lax.fori_loop(0, nrows, body, 0)


def _copy_span(src_at, dst_at, s):
    for off, sz in ((0, 128), (128, 128), (256, 128), (384, 128), (512, 112)):
        pltpu.sync_copy(src_at(off, sz), dst_at(off, sz))

    @pl.when(s == NS - 1)
    def _():
        pltpu.sync_copy(src_at(RPS, 16), dst_at(RPS, 16))


def _grp_ring(gbufs, p_hbm, acc, idx_at, didx, t0):
    def consume(i):
        gb, gs, ss = gbufs[i % NRING]
        pltpu.make_async_copy(p_hbm.at[idx_at(i)], gb, gs).wait()
        pltpu.async_copy(gb, acc.at[didx.at[i]], ss, add=True)

    for i in range(GSZ):
        gb, gs, ss = gbufs[i % NRING]

        @pl.when(t0 + i < NCHUNKS)
        def _(i=i, gb=gb, gs=gs, ss=ss):
            if i >= NRING:
                pltpu.make_async_copy(gb, acc.at[didx.at[i - NRING]], ss).wait()
            pltpu.async_copy(p_hbm.at[idx_at(i)], gb, gs)

        if i >= 2:
            @pl.when(t0 + i - 2 < NCHUNKS)
            def _(i=i):
                consume(i - 2)

    for i in (GSZ - 2, GSZ - 1):
        @pl.when(t0 + i < NCHUNKS)
        def _(i=i):
            consume(i)

    for b in range(NRING):
        gb, gs, ss = gbufs[b]

        @pl.when(t0 + b < NCHUNKS)
        def _(b=b, gb=gb, ss=ss):
            pltpu.make_async_copy(gb, acc.at[didx.at[b]], ss).wait()


@functools.lru_cache(maxsize=None)
def _make_edge_scatter(half):
    @functools.partial(
        pl.kernel,
        out_type=jax.ShapeDtypeStruct((NC * NN, half), jnp.float32),
        mesh=_mesh(),
        scratch_types=[
            pltpu.VMEM_SHARED((NN, half), jnp.float32),
            pltpu.VMEM((GSZ, CHUNK), jnp.int32),
            pltpu.VMEM((GSZ, CHUNK), jnp.int32),
            pltpu.VMEM((GSZ, CHUNK), jnp.int32),
            pltpu.VMEM((NRING * CHUNK, half), jnp.float32),
            pltpu.SemaphoreType.DMA,
            pltpu.SemaphoreType.DMA,
            pltpu.SemaphoreType.DMA,
            pltpu.SemaphoreType.DMA,
            pltpu.SemaphoreType.DMA,
            pltpu.SemaphoreType.DMA,
            pltpu.SemaphoreType.DMA,
            pltpu.SemaphoreType.DMA,
        ],
    )
    def k(p_hbm, src_hbm, dst_hbm, out_hbm, acc,
          sidx, didx, gidx, gring, gs0, gs1, gs2, gs3, ss0, ss1, ss2, ss3):
        c = lax.axis_index("c")
        s = lax.axis_index("s")
        gbufs = tuple(
            (gring.at[pl.ds(b * CHUNK, CHUNK)], gs, ss)
            for b, (gs, ss) in enumerate(((gs0, ss0), (gs1, ss1),
                                          (gs2, ss2), (gs3, ss3))))

        _zero_rows(gring, NRING * CHUNK, half)
        base_row = s * RPS
        _copy_span(
            lambda off, sz: gring.at[pl.ds(0, sz)],
            lambda off, sz: acc.at[pl.ds(base_row + off, sz)], s)
        plsc.subcore_barrier()

        off_gather = c * NN
        ngrp = 9 + jnp.where(s < NG - 9 * NS, 1, 0)

        def grp(j, _):
            g = s + j * NS
            pltpu.sync_copy(src_hbm.at[pl.ds(g * GSZ, GSZ)], sidx)
            pltpu.sync_copy(dst_hbm.at[pl.ds(g * GSZ, GSZ)], didx)
            for i in range(GSZ):
                for r in range(CHUNK // LANES):
                    sl = pl.ds(r * LANES, LANES)
                    gidx[i, sl] = sidx[i, sl] + off_gather
            _grp_ring(gbufs, p_hbm, acc, lambda i: gidx.at[i], didx, g * GSZ)
            return 0

        lax.fori_loop(0, ngrp, grp, 0)
        plsc.subcore_barrier()

        out_base = c * NN + s * RPS
        _copy_span(
            lambda off, sz: acc.at[pl.ds(base_row + off, sz)],
            lambda off, sz: out_hbm.at[pl.ds(out_base + off, sz)], s)

    return k


@functools.lru_cache(maxsize=None)
def _make_edge_scatter_esplit(width):
    @functools.partial(
        pl.kernel,
        out_type=jax.ShapeDtypeStruct((NC * NN, width), jnp.float32),
        mesh=_mesh(),
        scratch_types=[
            pltpu.VMEM_SHARED((NN, width), jnp.float32),
            pltpu.VMEM((GSZ, CHUNK), jnp.int32),
            pltpu.VMEM((GSZ, CHUNK), jnp.int32),
            pltpu.VMEM((NRING * CHUNK, width), jnp.float32),
            pltpu.SemaphoreType.DMA,
            pltpu.SemaphoreType.DMA,
            pltpu.SemaphoreType.DMA,
            pltpu.SemaphoreType.DMA,
            pltpu.SemaphoreType.DMA,
            pltpu.SemaphoreType.DMA,
            pltpu.SemaphoreType.DMA,
            pltpu.SemaphoreType.DMA,
        ],
    )
    def k(p_hbm, src_hbm, dst_hbm, out_hbm, acc,
          sidx, didx, gring, gs0, gs1, gs2, gs3, ss0, ss1, ss2, ss3):
        c = lax.axis_index("c")
        s = lax.axis_index("s")
        wid = c * NS + s
        gbufs = tuple(
            (gring.at[pl.ds(b * CHUNK, CHUNK)], gs, ss)
            for b, (gs, ss) in enumerate(((gs0, ss0), (gs1, ss1),
                                          (gs2, ss2), (gs3, ss3))))

        _zero_rows(gring, NRING * CHUNK, width)
        base_row = s * RPS
        _copy_span(
            lambda off, sz: gring.at[pl.ds(0, sz)],
            lambda off, sz: acc.at[pl.ds(base_row + off, sz)], s)
        plsc.subcore_barrier()

        ngrp = 4 + jnp.where(wid < NG - 4 * NC * NS, 1, 0)

        def grp(j, _):
            g = wid + j * NC * NS
            pltpu.sync_copy(src_hbm.at[pl.ds(g * GSZ, GSZ)], sidx)
            pltpu.sync_copy(dst_hbm.at[pl.ds(g * GSZ, GSZ)], didx)
            _grp_ring(gbufs, p_hbm, acc, lambda i: sidx.at[i], didx, g * GSZ)
            return 0

        lax.fori_loop(0, ngrp, grp, 0)
        plsc.subcore_barrier()

        out_base = c * NN + s * RPS
        _copy_span(
            lambda off, sz: acc.at[pl.ds(base_row + off, sz)],
            lambda off, sz: out_hbm.at[pl.ds(out_base + off, sz)], s)

    return k


@functools.lru_cache(maxsize=None)
def _make_deg_kernel():
    width = DEGW
    nw = NC * NS
    base = NCHUNKS // nw
    rem = NCHUNKS - base * nw
    niter = (base + (1 if rem else 0) + 1) // 2

    @functools.partial(
        pl.kernel,
        out_type=jax.ShapeDtypeStruct((NC * NN, width), jnp.float32),
        mesh=_mesh(),
        scratch_types=[
            pltpu.VMEM_SHARED((NN, width), jnp.float32),
            pltpu.VMEM((CHUNK,), jnp.int32),
            pltpu.VMEM((CHUNK,), jnp.int32),
            pltpu.VMEM((128, width), jnp.float32),
            pltpu.VMEM((CHUNK, width), jnp.float32),
            pltpu.SemaphoreType.DMA,
            pltpu.SemaphoreType.DMA,
        ],
    )
    def k(dst_hbm, out_hbm, acc, dstv0, dstv1, zerob, onesb, sem0, sem1):
        c = lax.axis_index("c")
        s = lax.axis_index("s")
        wid = c * NS + s
        bufs = ((dstv0, sem0), (dstv1, sem1))

        _zero_rows(zerob, 128, width)
        one = jnp.ones((LANES,), jnp.float32)

        def fill(r, _):
            for l in range(width // LANES):
                onesb[r, pl.ds(l * LANES, LANES)] = one
            return 0

        lax.fori_loop(0, CHUNK, fill, 0)

        base_row = s * RPS
        _copy_span(
            lambda off, sz: zerob.at[pl.ds(0, sz)],
            lambda off, sz: acc.at[pl.ds(base_row + off, sz)], s)
        plsc.subcore_barrier()

        nt = base + jnp.where(wid < rem, 1, 0)

        def body(j2, _):
            for b2 in (0, 1):
                kk = j2 * 2 + b2
                dv, sm = bufs[b2]

                @pl.when(kk < nt)
                def _():
                    @pl.when(kk >= 2)
                    def _():
                        pltpu.make_async_copy(onesb, acc.at[dv], sm).wait()

                    ebase = (wid + kk * NC * NS) * CHUNK
                    pltpu.sync_copy(dst_hbm.at[pl.ds(ebase, CHUNK)], dv)
                    pltpu.async_copy(onesb, acc.at[dv], sm, add=True)
            return 0

        lax.fori_loop(0, niter, body, 0)
        for b2 in (0, 1):
            dv, sm = bufs[b2]
            pltpu.make_async_copy(onesb, acc.at[dv], sm).wait()
        plsc.subcore_barrier()

        out_base = c * NN + s * RPS
        _copy_span(
            lambda off, sz: acc.at[pl.ds(base_row + off, sz)],
            lambda off, sz: out_hbm.at[pl.ds(out_base + off, sz)], s)

    return k


DEGW = 128


def _dinv_from_deg(deg_ref):
    dsum = deg_ref[0] + deg_ref[1]
    return lax.rsqrt(1.0 + dsum[:, 0:1])


def _split_out(out_ref, pn, half):
    out_ref[0, :, :] = pn[:, :half]
    out_ref[1, :, :] = pn[:, half:]


def _layer1_body(x_ref, deg_ref, w_ref, out_ref):
    dinv = _dinv_from_deg(deg_ref)
    m = jnp.dot(x_ref[...], w_ref[...], preferred_element_type=jnp.float32)
    _split_out(out_ref, dinv * m, HH // 2)


def _mid_body(s_ref, p_ref, deg_ref, b_ref, w_ref, out_ref, *, half_out):
    dinv = _dinv_from_deg(deg_ref)
    spp = jnp.concatenate(
        [s_ref[0] + p_ref[0], s_ref[1] + p_ref[1]], axis=-1)
    h = jnp.maximum(b_ref[...] + dinv * spp, 0.0)
    m = jnp.dot(h, w_ref[...], preferred_element_type=jnp.float32)
    pn = dinv * m
    if half_out is None:
        out_ref[...] = pn
    else:
        _split_out(out_ref, pn, half_out)


def _pool_body(s_ref, p_ref, deg_ref, b_ref, batch_ref, out_ref, cnt_ref):
    i = pl.program_id(0)

    @pl.when(i == 0)
    def _():
        out_ref[...] = jnp.zeros_like(out_ref)
        cnt_ref[...] = jnp.zeros_like(cnt_ref)

    dinv = _dinv_from_deg(deg_ref)
    spp = s_ref[0] + s_ref[1] + p_ref[...]
    h = b_ref[...] + dinv * spp
    bid = jnp.broadcast_to(batch_ref[0], (GG, ROWBLK))
    gid = lax.broadcasted_iota(jnp.int32, (GG, ROWBLK), 0)
    oh = (bid == gid).astype(jnp.float32)
    out_ref[...] += jnp.dot(oh, h, preferred_element_type=jnp.float32)
    cnt_ref[...] += jnp.broadcast_to(
        jnp.sum(oh, axis=1, keepdims=True), (GG, OO))

    @pl.when(i == NBLK - 1)
    def _():
        out_ref[...] = out_ref[...] / jnp.maximum(cnt_ref[...], 1.0)


def _rowspec(shape3):
    return pl.BlockSpec((shape3[0], ROWBLK, shape3[2]), lambda i: (0, i, 0))


def _full(shape):
    return pl.BlockSpec(shape, lambda i: tuple(0 for _ in shape))


_layer1 = pl.pallas_call(
    _layer1_body,
    grid=(NBLK,),
    in_specs=[
        pl.BlockSpec((ROWBLK, FF), lambda i: (i, 0)),
        _rowspec((NC, NN, DEGW)),
        _full((FF, HH)),
    ],
    out_specs=_rowspec((NC, NN, HH // 2)),
    out_shape=jax.ShapeDtypeStruct((NC, NN, HH // 2), jnp.float32),
)

_layer2 = pl.pallas_call(
    functools.partial(_mid_body, half_out=HH // 2),
    grid=(NBLK,),
    in_specs=[
        _rowspec((NC, NN, HH // 2)),
        _rowspec((NC, NN, HH // 2)),
        _rowspec((NC, NN, DEGW)),
        _full((1, HH)),
        _full((HH, HH)),
    ],
    out_specs=_rowspec((NC, NN, HH // 2)),
    out_shape=jax.ShapeDtypeStruct((NC, NN, HH // 2), jnp.float32),
)

_layer3 = pl.pallas_call(
    functools.partial(_mid_body, half_out=None),
    grid=(NBLK,),
    in_specs=[
        _rowspec((NC, NN, HH // 2)),
        _rowspec((NC, NN, HH // 2)),
        _rowspec((NC, NN, DEGW)),
        _full((1, HH)),
        _full((HH, OO)),
    ],
    out_specs=pl.BlockSpec((ROWBLK, OO), lambda i: (i, 0)),
    out_shape=jax.ShapeDtypeStruct((NN, OO), jnp.float32),
)

_pool = pl.pallas_call(
    _pool_body,
    grid=(NBLK,),
    in_specs=[
        _rowspec((NC, NN, OO)),
        pl.BlockSpec((ROWBLK, OO), lambda i: (i, 0)),
        _rowspec((NC, NN, DEGW)),
        _full((1, OO)),
        pl.BlockSpec((1, 1, ROWBLK), lambda i: (i, 0, 0)),
    ],
    out_specs=_full((GG, OO)),
    out_shape=jax.ShapeDtypeStruct((GG, OO), jnp.float32),
    scratch_shapes=[pltpu.VMEM((GG, OO), jnp.float32)],
)


def _pad_idx(a):
    pad = jnp.zeros((NIDX * CHUNK - EE,), jnp.int32)
    return jnp.concatenate([a, pad]).reshape(NIDX, CHUNK)


def kernel(x, edge_index, batch, W1, b1, W2, b2, W3, b3):
    src = edge_index[0]
    dst = edge_index[1]
    srcp = _pad_idx(src)
    dstp = _pad_idx(dst)
    scat128 = _make_edge_scatter(HH // 2)
    scat_last = _make_edge_scatter_esplit(OO)
    degs = _make_deg_kernel()(dst).reshape(NC, NN, DEGW)
    p1 = _layer1(x, degs, W1)
    s1 = scat128(p1.reshape(NC * NN, HH // 2), srcp, dstp).reshape(NC, NN, HH // 2)
    p2 = _layer2(s1, p1, degs, b1.reshape(1, HH), W2)
    s2 = scat128(p2.reshape(NC * NN, HH // 2), srcp, dstp).reshape(NC, NN, HH // 2)
    p3 = _layer3(s2, p2, degs, b2.reshape(1, HH), W3)
    s3 = scat_last(p3, srcp, dstp).reshape(NC, NN, OO)
    return _pool(s3, p3, degs, b3.reshape(1, OO), batch.reshape(NBLK, 1, ROWBLK))
